# baseline (speedup 1.0000x reference)
import functools
import jax
import jax.numpy as jnp
from jax import lax
from jax.experimental import pallas as pl
from jax.experimental.pallas import tpu as pltpu
from jax.experimental.pallas import tpu_sc as plsc

_L = 16
_NW = 32
_NB = 32
_NH = 200
_GPW = _NH * _NB // _NW


def _sc_embed(i4, table):
    mesh = plsc.VectorSubcoreMesh(core_axis_name="c", subcore_axis_name="s")

    @functools.partial(
        pl.kernel,
        out_type=jax.ShapeDtypeStruct((_NH, 8, _NB, 8, 128), jnp.float32),
        mesh=mesh,
        scratch_types=[
            [pltpu.VMEM((128,), jnp.int32) for _ in range(2)],
            [pltpu.VMEM((128, 64), jnp.float32) for _ in range(2)],
            [pltpu.VMEM((8, 8, 128), jnp.float32) for _ in range(2)],
            [pltpu.SemaphoreType.DMA for _ in range(2)],
            [pltpu.SemaphoreType.DMA for _ in range(2)],
        ],
        compiler_params=pltpu.CompilerParams(
            use_tc_tiling_on_sc=False, needs_layout_passes=False),
    )
    def k(t_hbm, i4_hbm, out_hbm, idx_v, rows_v, ot_v, gsem, osem):
        wid = lax.axis_index("s") * 2 + lax.axis_index("c")
        g0w = wid * _GPW

        def start_gather(g, slot):
            h = g // _NB
            bh = g % _NB
            pltpu.sync_copy(i4_hbm.at[h // 8, bh, h % 8], idx_v[slot])
            pltpu.async_copy(t_hbm.at[idx_v[slot]], rows_v[slot], gsem[slot])

        start_gather(g0w, 0)

        @pl.loop(0, _GPW, step=2)
        def _outer(gg):
            for b in range(2):
                g = g0w + gg + b
                slot = b
                nslot = 1 - b

                @pl.when(gg + b >= 1)
                def _():
                    gp = g - 1
                    pltpu.make_async_copy(
                        ot_v[nslot],
                        out_hbm.at[gp // _NB, :, gp % _NB],
                        osem[nslot]).wait()

                @pl.when(gg + b + 1 < _GPW)
                def _():
                    start_gather(g + 1, nslot)

                pltpu.make_async_copy(t_hbm.at[idx_v[slot]], rows_v[slot],
                                      gsem[slot]).wait()

                rows = rows_v[slot]
                ot = ot_v[slot]

                for c in range(8):
                    rvec = lax.iota(jnp.int32, _L) + (c * _L)

                    @plsc.parallel_loop(0, 64, unroll=4)
                    def _t(e):
                        cvec = jnp.broadcast_to(e, (_L,))
                        vals = plsc.load_gather(rows, [rvec, cvec])
                        ot[e // 8, e % 8, pl.ds(c * _L, _L)] = vals * 8.0

                pltpu.async_copy(ot, out_hbm.at[g // _NB, :, g % _NB],
                                 osem[slot])

        gl = g0w + _GPW - 1
        pltpu.make_async_copy(ot_v[1], out_hbm.at[gl // _NB, :, gl % _NB],
                              osem[1]).wait()

    return k(table, i4)


def kernel(input, table):
    i4 = jnp.transpose(input.reshape(32, 128, 25, 8), (2, 0, 3, 1))
    out5 = _sc_embed(i4, table)
    out = jnp.transpose(out5, (2, 4, 0, 1, 3)).reshape(4096, 200, 64)
    return out


# speedup vs baseline: 1.0440x; 1.0440x over previous
import functools
import jax
import jax.numpy as jnp
from jax import lax
from jax.experimental import pallas as pl
from jax.experimental.pallas import tpu as pltpu
from jax.experimental.pallas import tpu_sc as plsc

_L = 16
_NW = 32
_NB = 32
_NH = 200
_HB = 2
_GPW = (_NH // _HB) * _NB // _NW


def _sc_embed(i4, table):
    mesh = plsc.VectorSubcoreMesh(core_axis_name="c", subcore_axis_name="s")

    @functools.partial(
        pl.kernel,
        out_type=jax.ShapeDtypeStruct((_NH, 8, _NB, 8, 128), jnp.float32),
        mesh=mesh,
        scratch_types=[
            [pltpu.VMEM((_HB, 128), jnp.int32) for _ in range(2)],
            [pltpu.VMEM((_HB, 128, 64), jnp.float32) for _ in range(2)],
            [pltpu.VMEM((_HB, 8, 8, 128), jnp.float32) for _ in range(2)],
            [pltpu.SemaphoreType.DMA for _ in range(2)],
            [pltpu.SemaphoreType.DMA for _ in range(2)],
        ],
        compiler_params=pltpu.CompilerParams(
            use_tc_tiling_on_sc=False, needs_layout_passes=False),
    )
    def k(t_hbm, i4_hbm, out_hbm, idx_v, rows_v, ot_v, gsem, osem):
        wid = lax.axis_index("s") * 2 + lax.axis_index("c")
        g0w = wid * _GPW
        iotas = [lax.iota(jnp.int32, _L) + (c * _L) for c in range(8)]
        hivecs = [jnp.broadcast_to(jnp.int32(hi), (_L,)) for hi in range(_HB)]

        def start_gather(g, slot):
            hb = g // _NB
            bh = g % _NB
            h_base = hb * _HB
            pltpu.sync_copy(
                i4_hbm.at[h_base // 8, bh, pl.ds(h_base % 8, _HB)],
                idx_v[slot])
            for hi in range(_HB):
                pltpu.async_copy(t_hbm.at[idx_v[slot].at[hi]],
                                 rows_v[slot].at[hi], gsem[slot])

        start_gather(g0w, 0)

        @pl.loop(0, _GPW, step=2)
        def _outer(gg):
            for b in range(2):
                g = g0w + gg + b
                slot = b
                nslot = 1 - b

                @pl.when(gg + b >= 1)
                def _():
                    gp = g - 1
                    hp = (gp // _NB) * _HB
                    for hi in range(_HB):
                        pltpu.make_async_copy(
                            ot_v[nslot].at[hi],
                            out_hbm.at[hp + hi, :, gp % _NB],
                            osem[nslot]).wait()

                @pl.when(gg + b + 1 < _GPW)
                def _():
                    start_gather(g + 1, nslot)

                for hi in range(_HB):
                    pltpu.make_async_copy(t_hbm.at[idx_v[slot].at[hi]],
                                          rows_v[slot].at[hi],
                                          gsem[slot]).wait()

                rows = rows_v[slot]
                ot = ot_v[slot]

                for hi in range(_HB):
                    hivec = hivecs[hi]

                    @plsc.parallel_loop(0, 64, unroll=4)
                    def _t(e):
                        cvec = jnp.broadcast_to(e, (_L,))
                        for c in range(8):
                            vals = plsc.load_gather(
                                rows, [hivec, iotas[c], cvec])
                            ot[hi, e // 8, e % 8,
                               pl.ds(c * _L, _L)] = vals * 8.0

                hb4 = (g // _NB) * _HB
                for hi in range(_HB):
                    pltpu.async_copy(ot.at[hi],
                                     out_hbm.at[hb4 + hi, :, g % _NB],
                                     osem[slot])

        gl = g0w + _GPW - 1
        hl = (gl // _NB) * _HB
        for hi in range(_HB):
            pltpu.make_async_copy(ot_v[1].at[hi],
                                  out_hbm.at[hl + hi, :, gl % _NB],
                                  osem[1]).wait()

    return k(table, i4)


def kernel(input, table):
    i4 = jnp.transpose(input.reshape(32, 128, 25, 8), (2, 0, 3, 1))
    out5 = _sc_embed(i4, table)
    out = jnp.transpose(out5, (2, 4, 0, 1, 3)).reshape(4096, 200, 64)
    return out


# speedup vs baseline: 1.5218x; 1.4577x over previous
import functools
import jax
import jax.numpy as jnp
from jax import lax
from jax.experimental import pallas as pl
from jax.experimental.pallas import tpu as pltpu
from jax.experimental.pallas import tpu_sc as plsc

_D = 64
_L = 16
_NW = 32
_C = 512
_SCALE = 8.0


def _sc_embed(idx_flat, table):
    b_total = idx_flat.shape[0]
    b_per_w = b_total // _NW
    n_chunks = b_per_w // _C
    assert n_chunks % 2 == 0 and n_chunks * _C == b_per_w
    mesh = plsc.VectorSubcoreMesh(core_axis_name="c", subcore_axis_name="s")

    @functools.partial(
        pl.kernel,
        out_type=jax.ShapeDtypeStruct((b_total, 128), jnp.float32),
        mesh=mesh,
        scratch_types=[
            [pltpu.VMEM((_C,), jnp.int32) for _ in range(2)],
            [pltpu.VMEM((_C, _D), jnp.float32) for _ in range(2)],
            [pltpu.SemaphoreType.DMA for _ in range(2)],
            [pltpu.SemaphoreType.DMA for _ in range(2)],
        ],
        compiler_params=pltpu.CompilerParams(
            use_tc_tiling_on_sc=False, needs_layout_passes=False),
    )
    def k(table_hbm, idx_hbm, out_hbm, idx_v, rows_v, gsem, osem):
        wid = lax.axis_index("s") * 2 + lax.axis_index("c")
        base = wid * b_per_w

        def start_gather(g, slot):
            row0 = base + g * _C
            pltpu.sync_copy(idx_hbm.at[pl.ds(row0, _C)], idx_v[slot])
            pltpu.async_copy(table_hbm.at[idx_v[slot]], rows_v[slot],
                             gsem[slot])

        start_gather(0, 0)

        @pl.loop(0, n_chunks, step=2)
        def _outer(g0):
            for b in range(2):
                g = g0 + b
                slot = b
                nslot = 1 - b

                @pl.when(g >= 1)
                def _():
                    pltpu.make_async_copy(
                        rows_v[nslot],
                        out_hbm.at[pl.ds(base + (g - 1) * _C, _C),
                                   pl.ds(0, _D)],
                        osem[nslot]).wait()

                @pl.when(g + 1 < n_chunks)
                def _():
                    start_gather(g + 1, nslot)

                pltpu.make_async_copy(table_hbm.at[idx_v[slot]],
                                      rows_v[slot], gsem[slot]).wait()

                buf = rows_v[slot]

                @plsc.parallel_loop(0, _C, unroll=4)
                def _scale(r):
                    for c in range(0, _D, _L):
                        buf[r, pl.ds(c, _L)] = buf[r, pl.ds(c, _L)] * _SCALE

                pltpu.async_copy(
                    buf,
                    out_hbm.at[pl.ds(base + g * _C, _C), pl.ds(0, _D)],
                    osem[slot])

        pltpu.make_async_copy(
            rows_v[1],
            out_hbm.at[pl.ds(base + (n_chunks - 1) * _C, _C), pl.ds(0, _D)],
            osem[1]).wait()

    return k(table, idx_flat)


def kernel(input, table):
    b, h = input.shape
    idx_flat = input.reshape(b * h)
    out6 = _sc_embed(idx_flat, table)
    return out6[:, :_D].reshape(b, h, _D)


# speedup vs baseline: 1.5429x; 1.0139x over previous
import functools
import jax
import jax.numpy as jnp
from jax import lax
from jax.experimental import pallas as pl
from jax.experimental.pallas import tpu as pltpu
from jax.experimental.pallas import tpu_sc as plsc

_D = 64
_L = 16
_NW = 32
_C = 800
_SCALE = 8.0


def _sc_embed(idx_flat, table):
    b_total = idx_flat.shape[0]
    b_per_w = b_total // _NW
    n_chunks = b_per_w // _C
    assert n_chunks % 2 == 0 and n_chunks * _C == b_per_w
    mesh = plsc.VectorSubcoreMesh(core_axis_name="c", subcore_axis_name="s")

    @functools.partial(
        pl.kernel,
        out_type=jax.ShapeDtypeStruct((b_total, 128), jnp.float32),
        mesh=mesh,
        scratch_types=[
            [pltpu.VMEM((_C,), jnp.int32) for _ in range(2)],
            [pltpu.VMEM((_C, _D), jnp.float32) for _ in range(2)],
            [pltpu.SemaphoreType.DMA for _ in range(2)],
            [pltpu.SemaphoreType.DMA for _ in range(2)],
        ],
        compiler_params=pltpu.CompilerParams(
            use_tc_tiling_on_sc=False, needs_layout_passes=False),
    )
    def k(table_hbm, idx_hbm, out_hbm, idx_v, rows_v, gsem, osem):
        wid = lax.axis_index("s") * 2 + lax.axis_index("c")
        base = wid * b_per_w

        def start_gather(g, slot):
            row0 = base + g * _C
            pltpu.sync_copy(idx_hbm.at[pl.ds(row0, _C)], idx_v[slot])
            pltpu.async_copy(table_hbm.at[idx_v[slot]], rows_v[slot],
                             gsem[slot])

        start_gather(0, 0)

        @pl.loop(0, n_chunks, step=2)
        def _outer(g0):
            for b in range(2):
                g = g0 + b
                slot = b
                nslot = 1 - b

                @pl.when(g >= 1)
                def _():
                    pltpu.make_async_copy(
                        rows_v[nslot],
                        out_hbm.at[pl.ds(base + (g - 1) * _C, _C),
                                   pl.ds(0, _D)],
                        osem[nslot]).wait()

                @pl.when(g + 1 < n_chunks)
                def _():
                    start_gather(g + 1, nslot)

                pltpu.make_async_copy(table_hbm.at[idx_v[slot]],
                                      rows_v[slot], gsem[slot]).wait()

                buf = rows_v[slot]

                @plsc.parallel_loop(0, _C, unroll=4)
                def _scale(r):
                    for c in range(0, _D, _L):
                        buf[r, pl.ds(c, _L)] = buf[r, pl.ds(c, _L)] * _SCALE

                pltpu.async_copy(
                    buf,
                    out_hbm.at[pl.ds(base + g * _C, _C), pl.ds(0, _D)],
                    osem[slot])

        pltpu.make_async_copy(
            rows_v[1],
            out_hbm.at[pl.ds(base + (n_chunks - 1) * _C, _C), pl.ds(0, _D)],
            osem[1]).wait()

    return k(table, idx_flat)


def kernel(input, table):
    b, h = input.shape
    idx_flat = input.reshape(b * h)
    out6 = _sc_embed(idx_flat, table)
    return out6[:, :_D].reshape(b, h, _D)
